# Initial kernel scaffold; baseline (speedup 1.0000x reference)
import jax
import jax.numpy as jnp
from jax.experimental import pallas as pl


def kernel(x, edge_index, W1, a1_src, a1_dst, b1, W2, a2_src, a2_dst, b2, Wfc, bfc):
    raise NotImplementedError("write your pallas kernel here")


# speedup vs baseline: 56.8907x; 56.8907x over previous
import functools

import jax
import jax.numpy as jnp
from jax import lax
from jax.experimental import pallas as pl
from jax.experimental.pallas import tpu as pltpu
from jax.experimental.pallas import tpu_sc as plsc

N = 10000
E = 320000
D_IN = 128
D1 = 16
D2 = 8
DP = 16

NC = 2
NS = 16
NW = NC * NS
L = 16

NG = 8
NQ = 4
DQ = DP // NQ
EPG = E // NG
CHK = 2000
NCHKS = EPG // CHK
NPAD = 10240


def _leaky(t):
    return jnp.where(t >= 0.0, t, 0.2 * t)


def _sc_edge(src, dst, a_s, a_d, m16, hT4):
    mesh = plsc.VectorSubcoreMesh(core_axis_name="c", subcore_axis_name="s",
                                  num_cores=NC, num_subcores=NS)

    @functools.partial(
        pl.kernel,
        out_type=(
            jax.ShapeDtypeStruct((NW, DQ, NPAD), jnp.float32),
            jax.ShapeDtypeStruct((NG, 1, NPAD), jnp.float32),
        ),
        mesh=mesh,
        compiler_params=pltpu.CompilerParams(needs_layout_passes=False),
        scratch_types=[
            pltpu.VMEM((NPAD,), jnp.float32),
            pltpu.VMEM((NPAD,), jnp.float32),
            pltpu.VMEM((L,), jnp.float32),
            pltpu.VMEM((DQ, NPAD), jnp.float32),
            pltpu.VMEM((DQ, NPAD), jnp.float32),
            pltpu.VMEM((1, NPAD), jnp.float32),
            pltpu.VMEM((CHK,), jnp.int32),
            pltpu.VMEM((CHK,), jnp.int32),
        ],
    )
    def edge_kernel(src_hbm, dst_hbm, as_hbm, ad_hbm, m_hbm, h_hbm,
                    out_hbm, den_hbm,
                    as_v, ad_v, m_v, hq_v, acc_v, den_v, sv, dv):
        cid = lax.axis_index("c")
        sid = lax.axis_index("s")
        wid = cid * NS + sid
        q = sid % NQ
        gg = cid * NQ + sid // NQ

        pltpu.sync_copy(as_hbm, as_v)
        pltpu.sync_copy(ad_hbm, ad_v)
        pltpu.sync_copy(m_hbm, m_v)
        pltpu.sync_copy(h_hbm.at[q], hq_v)

        zeros = jnp.zeros((L,), jnp.float32)

        @pl.loop(0, NPAD // L)
        def _zd(i):
            den_v[0, pl.ds(i * L, L)] = zeros

        @pl.loop(0, DQ)
        def _zr(r):
            @pl.loop(0, NPAD // L)
            def _zc(i):
                acc_v[r, pl.ds(i * L, L)] = zeros

        m_vec = m_v[...]
        row_ids = [jnp.full((L,), r, jnp.int32) for r in range(DQ)]
        zero_ids = jnp.zeros((L,), jnp.int32)

        base = gg * EPG

        @pl.loop(0, NCHKS)
        def _chunk(c):
            pltpu.sync_copy(src_hbm.at[pl.ds(base + c * CHK, CHK)], sv)
            pltpu.sync_copy(dst_hbm.at[pl.ds(base + c * CHK, CHK)], dv)

            @pl.loop(0, CHK // L)
            def _vec(i):
                s_idx = sv[pl.ds(i * L, L)]
                d_idx = dv[pl.ds(i * L, L)]
                e = (plsc.load_gather(as_v, [s_idx])
                     + plsc.load_gather(ad_v, [d_idx]))
                w = jnp.exp(_leaky(e) - m_vec)

                @pl.when(q == 0)
                def _den():
                    plsc.addupdate_scatter(den_v, [zero_ids, d_idx], w)

                for r in range(DQ):
                    hv = plsc.load_gather(hq_v, [row_ids[r], s_idx])
                    plsc.addupdate_scatter(acc_v, [row_ids[r], d_idx], hv * w)

        pltpu.sync_copy(acc_v, out_hbm.at[wid])

        @pl.when(q == 0)
        def _pub_den():
            pltpu.sync_copy(den_v, den_hbm.at[gg])

    return edge_kernel(src, dst, a_s, a_d, m16, hT4)


def _tc1(xp, W1, a1s, a1d):
    def body(x_ref, w_ref, as_ref, ad_ref, h_ref, s_ref, d_ref, m_ref):
        h = jnp.dot(x_ref[...], w_ref[...], preferred_element_type=jnp.float32)
        hT = h.T
        h_ref[...] = hT
        a_s = jnp.dot(as_ref[...], hT, preferred_element_type=jnp.float32)
        a_d = jnp.dot(ad_ref[...], hT, preferred_element_type=jnp.float32)
        s_ref[...] = a_s
        d_ref[...] = a_d
        m_ref[...] = (jnp.max(a_s) + jnp.max(a_d)).reshape(1, 1)

    return pl.pallas_call(
        body,
        out_shape=(
            jax.ShapeDtypeStruct((D1, NPAD), jnp.float32),
            jax.ShapeDtypeStruct((1, NPAD), jnp.float32),
            jax.ShapeDtypeStruct((1, NPAD), jnp.float32),
            jax.ShapeDtypeStruct((1, 1), jnp.float32),
        ),
    )(xp, W1, a1s, a1d)


def _tc2(outp, denp, h1T, a_s1, a_d1, m1, b1, W2T, a2s, a2d):
    def body(o_ref, de_ref, h1_ref, s1_ref, d1_ref, m1_ref, b1_ref, w2_ref,
             as_ref, ad_ref, h2_ref, s_ref, d_ref, m_ref):
        w_self = jnp.exp(_leaky(s1_ref[...] + d1_ref[...]) - m1_ref[0, 0])
        op = o_ref[...].reshape(NC, NQ, NQ, DQ, NPAD)
        outT = op.sum(axis=(0, 1)).reshape(DP, NPAD)
        den = de_ref[...].sum(axis=0) + w_self + 1e-16
        numT = outT[:D1] + w_self * h1_ref[...]
        h1fT = jax.nn.relu(numT / den + b1_ref[...])
        h2T = jnp.dot(w2_ref[...], h1fT, preferred_element_type=jnp.float32)
        h2_ref[...] = jnp.concatenate(
            [h2T, jnp.zeros((DP - D2, NPAD), jnp.float32)], axis=0)
        a_s = jnp.dot(as_ref[...], h2T, preferred_element_type=jnp.float32)
        a_d = jnp.dot(ad_ref[...], h2T, preferred_element_type=jnp.float32)
        s_ref[...] = a_s
        d_ref[...] = a_d
        m_ref[...] = (jnp.max(a_s) + jnp.max(a_d)).reshape(1, 1)

    return pl.pallas_call(
        body,
        out_shape=(
            jax.ShapeDtypeStruct((DP, NPAD), jnp.float32),
            jax.ShapeDtypeStruct((1, NPAD), jnp.float32),
            jax.ShapeDtypeStruct((1, NPAD), jnp.float32),
            jax.ShapeDtypeStruct((1, 1), jnp.float32),
        ),
    )(outp, denp, h1T, a_s1, a_d1, m1, b1, W2T, a2s, a2d)


def _tc3(outp, denp, h2T, a_s2, a_d2, m2, b2, WfcT, bfc):
    def body(o_ref, de_ref, h2_ref, s2_ref, d2_ref, m2_ref, b2_ref, wf_ref,
             bf_ref, out_ref):
        w_self = jnp.exp(_leaky(s2_ref[...] + d2_ref[...]) - m2_ref[0, 0])
        op = o_ref[...].reshape(NC, NQ, NQ, DQ, NPAD)
        outT = op.sum(axis=(0, 1)).reshape(DP, NPAD)
        den = de_ref[...].sum(axis=0) + w_self + 1e-16
        numT = outT[:D2] + w_self * h2_ref[:D2, :]
        h2fT = jax.nn.relu(numT / den + b2_ref[...])
        out_ref[...] = jnp.dot(wf_ref[...], h2fT,
                               preferred_element_type=jnp.float32) + bf_ref[0, 0]

    return pl.pallas_call(
        body,
        out_shape=jax.ShapeDtypeStruct((1, NPAD), jnp.float32),
    )(outp, denp, h2T, a_s2, a_d2, m2, b2, WfcT, bfc)


def kernel(x, edge_index, W1, a1_src, a1_dst, b1, W2, a2_src, a2_dst, b2,
           Wfc, bfc):
    ei = edge_index.astype(jnp.int32)
    src = ei[0]
    dst = ei[1]
    xp = jnp.pad(x, ((0, NPAD - N), (0, 0)))

    h1T, a_s1, a_d1, m1 = _tc1(xp, W1, a1_src.reshape(1, D1),
                               a1_dst.reshape(1, D1))
    m1_16 = jnp.broadcast_to(m1.reshape(()), (L,))
    outp1, denp1 = _sc_edge(src, dst, a_s1.reshape(NPAD), a_d1.reshape(NPAD),
                            m1_16, h1T.reshape(NQ, DQ, NPAD))

    h2T, a_s2, a_d2, m2 = _tc2(outp1, denp1, h1T, a_s1, a_d1, m1,
                               b1.reshape(D1, 1), W2.T,
                               a2_src.reshape(1, D2), a2_dst.reshape(1, D2))
    m2_16 = jnp.broadcast_to(m2.reshape(()), (L,))
    outp2, denp2 = _sc_edge(src, dst, a_s2.reshape(NPAD), a_d2.reshape(NPAD),
                            m2_16, h2T.reshape(NQ, DQ, NPAD))

    out = _tc3(outp2, denp2, h2T, a_s2, a_d2, m2, b2.reshape(D2, 1),
               Wfc.T, bfc.reshape(1, 1))
    return out.reshape(NPAD, 1)[:N]


# speedup vs baseline: 60.3740x; 1.0612x over previous
import functools

import jax
import jax.numpy as jnp
from jax import lax
from jax.experimental import pallas as pl
from jax.experimental.pallas import tpu as pltpu
from jax.experimental.pallas import tpu_sc as plsc

N = 10000
E = 320000
D_IN = 128
D1 = 16
D2 = 8
DP = 16

NC = 2
NS = 16
NW = NC * NS
L = 16

NG = 8
NQ = 4
DQ = DP // NQ
EPG = E // NG
CHK = 2000
NCHKS = EPG // CHK
NPAD = 10240


def _leaky(t):
    return jnp.where(t >= 0.0, t, 0.2 * t)


def _sc_edge(src, dst, a_s, a_d, m16, hT4):
    mesh = plsc.VectorSubcoreMesh(core_axis_name="c", subcore_axis_name="s",
                                  num_cores=NC, num_subcores=NS)

    @functools.partial(
        pl.kernel,
        out_type=(
            jax.ShapeDtypeStruct((NW, DQ, NPAD), jnp.float32),
            jax.ShapeDtypeStruct((NG, 1, NPAD), jnp.float32),
        ),
        mesh=mesh,
        compiler_params=pltpu.CompilerParams(needs_layout_passes=False),
        scratch_types=[
            pltpu.VMEM((NPAD,), jnp.float32),
            pltpu.VMEM((NPAD,), jnp.float32),
            pltpu.VMEM((L,), jnp.float32),
            pltpu.VMEM((DQ, NPAD), jnp.float32),
            pltpu.VMEM((DQ, NPAD), jnp.float32),
            pltpu.VMEM((1, NPAD), jnp.float32),
            pltpu.VMEM((CHK,), jnp.int32),
            pltpu.VMEM((CHK,), jnp.int32),
        ],
    )
    def edge_kernel(src_hbm, dst_hbm, as_hbm, ad_hbm, m_hbm, h_hbm,
                    out_hbm, den_hbm,
                    as_v, ad_v, m_v, hq_v, acc_v, den_v, sv, dv):
        cid = lax.axis_index("c")
        sid = lax.axis_index("s")
        wid = cid * NS + sid
        q = sid % NQ
        gg = cid * NQ + sid // NQ

        pltpu.sync_copy(as_hbm, as_v)
        pltpu.sync_copy(ad_hbm, ad_v)
        pltpu.sync_copy(m_hbm, m_v)
        pltpu.sync_copy(h_hbm.at[q], hq_v)

        zeros = jnp.zeros((L,), jnp.float32)

        @pl.loop(0, NPAD // L, unroll=8)
        def _zd(i):
            den_v[0, pl.ds(i * L, L)] = zeros

        for r in range(DQ):
            @pl.loop(0, NPAD // L, unroll=8)
            def _zc(i, r=r):
                acc_v[r, pl.ds(i * L, L)] = zeros

        m_vec = m_v[...]
        row_ids = [jnp.full((L,), r, jnp.int32) for r in range(DQ)]
        zero_ids = jnp.zeros((L,), jnp.int32)

        base = gg * EPG

        @pl.loop(0, NCHKS)
        def _chunk(c):
            pltpu.sync_copy(src_hbm.at[pl.ds(base + c * CHK, CHK)], sv)
            pltpu.sync_copy(dst_hbm.at[pl.ds(base + c * CHK, CHK)], dv)

            @pl.loop(0, CHK // L, unroll=4)
            def _vec(i):
                s_idx = sv[pl.ds(i * L, L)]
                d_idx = dv[pl.ds(i * L, L)]
                e = (plsc.load_gather(as_v, [s_idx])
                     + plsc.load_gather(ad_v, [d_idx]))
                w = jnp.exp(_leaky(e) - m_vec)

                @pl.when(q == 0)
                def _den():
                    plsc.addupdate_scatter(den_v, [zero_ids, d_idx], w)

                for r in range(DQ):
                    hv = plsc.load_gather(hq_v, [row_ids[r], s_idx])
                    plsc.addupdate_scatter(acc_v, [row_ids[r], d_idx], hv * w)

        pltpu.sync_copy(acc_v, out_hbm.at[wid])

        @pl.when(q == 0)
        def _pub_den():
            pltpu.sync_copy(den_v, den_hbm.at[gg])

    return edge_kernel(src, dst, a_s, a_d, m16, hT4)


def _tc1(xp, W1, a1s, a1d):
    def body(x_ref, w_ref, as_ref, ad_ref, h_ref, s_ref, d_ref, m_ref):
        h = jnp.dot(x_ref[...], w_ref[...], preferred_element_type=jnp.float32)
        hT = h.T
        h_ref[...] = hT
        a_s = jnp.dot(as_ref[...], hT, preferred_element_type=jnp.float32)
        a_d = jnp.dot(ad_ref[...], hT, preferred_element_type=jnp.float32)
        s_ref[...] = a_s
        d_ref[...] = a_d
        m_ref[...] = (jnp.max(a_s) + jnp.max(a_d)).reshape(1, 1)

    return pl.pallas_call(
        body,
        out_shape=(
            jax.ShapeDtypeStruct((D1, NPAD), jnp.float32),
            jax.ShapeDtypeStruct((1, NPAD), jnp.float32),
            jax.ShapeDtypeStruct((1, NPAD), jnp.float32),
            jax.ShapeDtypeStruct((1, 1), jnp.float32),
        ),
    )(xp, W1, a1s, a1d)


def _tc2(outp, denp, h1T, a_s1, a_d1, m1, b1, W2T, a2s, a2d):
    def body(o_ref, de_ref, h1_ref, s1_ref, d1_ref, m1_ref, b1_ref, w2_ref,
             as_ref, ad_ref, h2_ref, s_ref, d_ref, m_ref):
        w_self = jnp.exp(_leaky(s1_ref[...] + d1_ref[...]) - m1_ref[0, 0])
        op = o_ref[...].reshape(NC, NQ, NQ, DQ, NPAD)
        outT = op.sum(axis=(0, 1)).reshape(DP, NPAD)
        den = de_ref[...].sum(axis=0) + w_self + 1e-16
        numT = outT[:D1] + w_self * h1_ref[...]
        h1fT = jax.nn.relu(numT / den + b1_ref[...])
        h2T = jnp.dot(w2_ref[...], h1fT, preferred_element_type=jnp.float32)
        h2_ref[...] = jnp.concatenate(
            [h2T, jnp.zeros((DP - D2, NPAD), jnp.float32)], axis=0)
        a_s = jnp.dot(as_ref[...], h2T, preferred_element_type=jnp.float32)
        a_d = jnp.dot(ad_ref[...], h2T, preferred_element_type=jnp.float32)
        s_ref[...] = a_s
        d_ref[...] = a_d
        m_ref[...] = (jnp.max(a_s) + jnp.max(a_d)).reshape(1, 1)

    return pl.pallas_call(
        body,
        out_shape=(
            jax.ShapeDtypeStruct((DP, NPAD), jnp.float32),
            jax.ShapeDtypeStruct((1, NPAD), jnp.float32),
            jax.ShapeDtypeStruct((1, NPAD), jnp.float32),
            jax.ShapeDtypeStruct((1, 1), jnp.float32),
        ),
    )(outp, denp, h1T, a_s1, a_d1, m1, b1, W2T, a2s, a2d)


def _tc3(outp, denp, h2T, a_s2, a_d2, m2, b2, WfcT, bfc):
    def body(o_ref, de_ref, h2_ref, s2_ref, d2_ref, m2_ref, b2_ref, wf_ref,
             bf_ref, out_ref):
        w_self = jnp.exp(_leaky(s2_ref[...] + d2_ref[...]) - m2_ref[0, 0])
        op = o_ref[...].reshape(NC, NQ, NQ, DQ, NPAD)
        outT = op.sum(axis=(0, 1)).reshape(DP, NPAD)
        den = de_ref[...].sum(axis=0) + w_self + 1e-16
        numT = outT[:D2] + w_self * h2_ref[:D2, :]
        h2fT = jax.nn.relu(numT / den + b2_ref[...])
        out_ref[...] = jnp.dot(wf_ref[...], h2fT,
                               preferred_element_type=jnp.float32) + bf_ref[0, 0]

    return pl.pallas_call(
        body,
        out_shape=jax.ShapeDtypeStruct((1, NPAD), jnp.float32),
    )(outp, denp, h2T, a_s2, a_d2, m2, b2, WfcT, bfc)


def kernel(x, edge_index, W1, a1_src, a1_dst, b1, W2, a2_src, a2_dst, b2,
           Wfc, bfc):
    ei = edge_index.astype(jnp.int32)
    src = ei[0]
    dst = ei[1]
    xp = jnp.pad(x, ((0, NPAD - N), (0, 0)))

    h1T, a_s1, a_d1, m1 = _tc1(xp, W1, a1_src.reshape(1, D1),
                               a1_dst.reshape(1, D1))
    m1_16 = jnp.broadcast_to(m1.reshape(()), (L,))
    outp1, denp1 = _sc_edge(src, dst, a_s1.reshape(NPAD), a_d1.reshape(NPAD),
                            m1_16, h1T.reshape(NQ, DQ, NPAD))

    h2T, a_s2, a_d2, m2 = _tc2(outp1, denp1, h1T, a_s1, a_d1, m1,
                               b1.reshape(D1, 1), W2.T,
                               a2_src.reshape(1, D2), a2_dst.reshape(1, D2))
    m2_16 = jnp.broadcast_to(m2.reshape(()), (L,))
    outp2, denp2 = _sc_edge(src, dst, a_s2.reshape(NPAD), a_d2.reshape(NPAD),
                            m2_16, h2T.reshape(NQ, DQ, NPAD))

    out = _tc3(outp2, denp2, h2T, a_s2, a_d2, m2, b2.reshape(D2, 1),
               Wfc.T, bfc.reshape(1, 1))
    return out.reshape(NPAD, 1)[:N]


# speedup vs baseline: 107.4959x; 1.7805x over previous
import functools

import jax
import jax.numpy as jnp
from jax import lax
from jax.experimental import pallas as pl
from jax.experimental.pallas import tpu as pltpu
from jax.experimental.pallas import tpu_sc as plsc

N = 10000
E = 320000
D_IN = 128
D1 = 16
D2 = 8
DP = 16

NC = 2
NS = 16
NW = NC * NS
L = 16

NG = 8
NQ = 4
DQ = DP // NQ
EPG = E // NG
CHK = 2000
NCHKS = EPG // CHK
NPAD = 10240


def _leaky(t):
    return jnp.where(t >= 0.0, t, 0.2 * t)


def _sc_edge(src, dst, a_s, a_d, m16, hT4):
    mesh = plsc.VectorSubcoreMesh(core_axis_name="c", subcore_axis_name="s",
                                  num_cores=NC, num_subcores=NS)

    @functools.partial(
        pl.kernel,
        out_type=(
            jax.ShapeDtypeStruct((NW, DQ, NPAD), jnp.float32),
            jax.ShapeDtypeStruct((NG, 1, NPAD), jnp.float32),
        ),
        mesh=mesh,
        compiler_params=pltpu.CompilerParams(needs_layout_passes=False),
        scratch_types=[
            pltpu.VMEM((NPAD,), jnp.float32),
            pltpu.VMEM((NPAD,), jnp.float32),
            pltpu.VMEM((L,), jnp.float32),
            pltpu.VMEM((DQ, NPAD), jnp.float32),
            pltpu.VMEM((DQ, NPAD), jnp.float32),
            pltpu.VMEM((1, NPAD), jnp.float32),
            pltpu.VMEM((CHK,), jnp.int32),
            pltpu.VMEM((CHK,), jnp.int32),
        ],
    )
    def edge_kernel(src_hbm, dst_hbm, as_hbm, ad_hbm, m_hbm, h_hbm,
                    out_hbm, den_hbm,
                    as_v, ad_v, m_v, hq_v, acc_v, den_v, sv, dv):
        cid = lax.axis_index("c")
        sid = lax.axis_index("s")
        wid = cid * NS + sid
        q = sid % NQ
        gg = cid * NQ + sid // NQ

        pltpu.sync_copy(as_hbm, as_v)
        pltpu.sync_copy(ad_hbm, ad_v)
        pltpu.sync_copy(m_hbm, m_v)
        pltpu.sync_copy(h_hbm.at[q], hq_v)

        zeros = jnp.zeros((L,), jnp.float32)

        @pl.loop(0, NPAD // L, unroll=8)
        def _zd(i):
            den_v[0, pl.ds(i * L, L)] = zeros

        for r in range(DQ):
            @pl.loop(0, NPAD // L, unroll=8)
            def _zc(i, r=r):
                acc_v[r, pl.ds(i * L, L)] = zeros

        m_vec = m_v[...]
        row_ids = [jnp.full((L,), r, jnp.int32) for r in range(DQ)]
        zero_ids = jnp.zeros((L,), jnp.int32)

        base = gg * EPG

        @pl.loop(0, NCHKS)
        def _chunk(c):
            pltpu.sync_copy(src_hbm.at[pl.ds(base + c * CHK, CHK)], sv)
            pltpu.sync_copy(dst_hbm.at[pl.ds(base + c * CHK, CHK)], dv)

            @plsc.parallel_loop(0, CHK // L, unroll=4)
            def _vec(i):
                s_idx = sv[pl.ds(i * L, L)]
                d_idx = dv[pl.ds(i * L, L)]
                e = (plsc.load_gather(as_v, [s_idx])
                     + plsc.load_gather(ad_v, [d_idx]))
                w = jnp.exp(_leaky(e) - m_vec)

                @pl.when(q == 0)
                def _den():
                    plsc.addupdate_scatter(den_v, [zero_ids, d_idx], w)

                for r in range(DQ):
                    hv = plsc.load_gather(hq_v, [row_ids[r], s_idx])
                    plsc.addupdate_scatter(acc_v, [row_ids[r], d_idx], hv * w)

        pltpu.sync_copy(acc_v, out_hbm.at[wid])

        @pl.when(q == 0)
        def _pub_den():
            pltpu.sync_copy(den_v, den_hbm.at[gg])

    return edge_kernel(src, dst, a_s, a_d, m16, hT4)


def _tc1(xp, W1, a1s, a1d):
    def body(x_ref, w_ref, as_ref, ad_ref, h_ref, s_ref, d_ref, m_ref):
        h = jnp.dot(x_ref[...], w_ref[...], preferred_element_type=jnp.float32)
        hT = h.T
        h_ref[...] = hT
        a_s = jnp.dot(as_ref[...], hT, preferred_element_type=jnp.float32)
        a_d = jnp.dot(ad_ref[...], hT, preferred_element_type=jnp.float32)
        s_ref[...] = a_s
        d_ref[...] = a_d
        m_ref[...] = (jnp.max(a_s) + jnp.max(a_d)).reshape(1, 1)

    return pl.pallas_call(
        body,
        out_shape=(
            jax.ShapeDtypeStruct((D1, NPAD), jnp.float32),
            jax.ShapeDtypeStruct((1, NPAD), jnp.float32),
            jax.ShapeDtypeStruct((1, NPAD), jnp.float32),
            jax.ShapeDtypeStruct((1, 1), jnp.float32),
        ),
    )(xp, W1, a1s, a1d)


def _tc2(outp, denp, h1T, a_s1, a_d1, m1, b1, W2T, a2s, a2d):
    def body(o_ref, de_ref, h1_ref, s1_ref, d1_ref, m1_ref, b1_ref, w2_ref,
             as_ref, ad_ref, h2_ref, s_ref, d_ref, m_ref):
        w_self = jnp.exp(_leaky(s1_ref[...] + d1_ref[...]) - m1_ref[0, 0])
        op = o_ref[...].reshape(NC, NQ, NQ, DQ, NPAD)
        outT = op.sum(axis=(0, 1)).reshape(DP, NPAD)
        den = de_ref[...].sum(axis=0) + w_self + 1e-16
        numT = outT[:D1] + w_self * h1_ref[...]
        h1fT = jax.nn.relu(numT / den + b1_ref[...])
        h2T = jnp.dot(w2_ref[...], h1fT, preferred_element_type=jnp.float32)
        h2_ref[...] = jnp.concatenate(
            [h2T, jnp.zeros((DP - D2, NPAD), jnp.float32)], axis=0)
        a_s = jnp.dot(as_ref[...], h2T, preferred_element_type=jnp.float32)
        a_d = jnp.dot(ad_ref[...], h2T, preferred_element_type=jnp.float32)
        s_ref[...] = a_s
        d_ref[...] = a_d
        m_ref[...] = (jnp.max(a_s) + jnp.max(a_d)).reshape(1, 1)

    return pl.pallas_call(
        body,
        out_shape=(
            jax.ShapeDtypeStruct((DP, NPAD), jnp.float32),
            jax.ShapeDtypeStruct((1, NPAD), jnp.float32),
            jax.ShapeDtypeStruct((1, NPAD), jnp.float32),
            jax.ShapeDtypeStruct((1, 1), jnp.float32),
        ),
    )(outp, denp, h1T, a_s1, a_d1, m1, b1, W2T, a2s, a2d)


def _tc3(outp, denp, h2T, a_s2, a_d2, m2, b2, WfcT, bfc):
    def body(o_ref, de_ref, h2_ref, s2_ref, d2_ref, m2_ref, b2_ref, wf_ref,
             bf_ref, out_ref):
        w_self = jnp.exp(_leaky(s2_ref[...] + d2_ref[...]) - m2_ref[0, 0])
        op = o_ref[...].reshape(NC, NQ, NQ, DQ, NPAD)
        outT = op.sum(axis=(0, 1)).reshape(DP, NPAD)
        den = de_ref[...].sum(axis=0) + w_self + 1e-16
        numT = outT[:D2] + w_self * h2_ref[:D2, :]
        h2fT = jax.nn.relu(numT / den + b2_ref[...])
        out_ref[...] = jnp.dot(wf_ref[...], h2fT,
                               preferred_element_type=jnp.float32) + bf_ref[0, 0]

    return pl.pallas_call(
        body,
        out_shape=jax.ShapeDtypeStruct((1, NPAD), jnp.float32),
    )(outp, denp, h2T, a_s2, a_d2, m2, b2, WfcT, bfc)


def kernel(x, edge_index, W1, a1_src, a1_dst, b1, W2, a2_src, a2_dst, b2,
           Wfc, bfc):
    ei = edge_index.astype(jnp.int32)
    src = ei[0]
    dst = ei[1]
    xp = jnp.pad(x, ((0, NPAD - N), (0, 0)))

    h1T, a_s1, a_d1, m1 = _tc1(xp, W1, a1_src.reshape(1, D1),
                               a1_dst.reshape(1, D1))
    m1_16 = jnp.broadcast_to(m1.reshape(()), (L,))
    outp1, denp1 = _sc_edge(src, dst, a_s1.reshape(NPAD), a_d1.reshape(NPAD),
                            m1_16, h1T.reshape(NQ, DQ, NPAD))

    h2T, a_s2, a_d2, m2 = _tc2(outp1, denp1, h1T, a_s1, a_d1, m1,
                               b1.reshape(D1, 1), W2.T,
                               a2_src.reshape(1, D2), a2_dst.reshape(1, D2))
    m2_16 = jnp.broadcast_to(m2.reshape(()), (L,))
    outp2, denp2 = _sc_edge(src, dst, a_s2.reshape(NPAD), a_d2.reshape(NPAD),
                            m2_16, h2T.reshape(NQ, DQ, NPAD))

    out = _tc3(outp2, denp2, h2T, a_s2, a_d2, m2, b2.reshape(D2, 1),
               Wfc.T, bfc.reshape(1, 1))
    return out.reshape(NPAD, 1)[:N]


# speedup vs baseline: 148.9507x; 1.3856x over previous
import functools

import jax
import jax.numpy as jnp
from jax import lax
from jax.experimental import pallas as pl
from jax.experimental.pallas import tpu as pltpu
from jax.experimental.pallas import tpu_sc as plsc

N = 10000
E = 320000
D_IN = 128
D1 = 16
D2 = 8
DP = 16

NC = 2
NS = 16
NW = NC * NS
L = 16

NG = 8
NQ = 4
DQ = DP // NQ
EPG = E // NG
CHK = 2000
NCHKS = EPG // CHK
NPAD = 10240


def _leaky(t):
    return jnp.where(t >= 0.0, t, 0.2 * t)


def _sc_edge(epk, a_s, a_d, m16, hT4):
    mesh = plsc.VectorSubcoreMesh(core_axis_name="c", subcore_axis_name="s",
                                  num_cores=NC, num_subcores=NS)

    @functools.partial(
        pl.kernel,
        out_type=(
            jax.ShapeDtypeStruct((NW, DQ, NPAD), jnp.float32),
            jax.ShapeDtypeStruct((NG, 1, NPAD), jnp.float32),
        ),
        mesh=mesh,
        compiler_params=pltpu.CompilerParams(needs_layout_passes=False),
        scratch_types=[
            pltpu.VMEM((NPAD,), jnp.float32),
            pltpu.VMEM((NPAD,), jnp.float32),
            pltpu.VMEM((L,), jnp.float32),
            pltpu.VMEM((DQ, NPAD), jnp.float32),
            pltpu.VMEM((DQ, NPAD), jnp.float32),
            pltpu.VMEM((1, NPAD), jnp.float32),
            pltpu.VMEM((CHK,), jnp.int32),
            pltpu.VMEM((CHK,), jnp.int32),
            pltpu.SemaphoreType.DMA,
            pltpu.SemaphoreType.DMA,
            pltpu.SemaphoreType.DMA,
        ],
    )
    def edge_kernel(ep_hbm, as_hbm, ad_hbm, m_hbm, h_hbm,
                    out_hbm, den_hbm,
                    as_v, ad_v, m_v, hq_v, acc_v, den_v, ev0, ev1,
                    sem_st, sem0, sem1):
        cid = lax.axis_index("c")
        sid = lax.axis_index("s")
        wid = cid * NS + sid
        q = sid % NQ
        gg = cid * NQ + sid // NQ
        base = gg * EPG

        pltpu.async_copy(as_hbm, as_v, sem_st)
        pltpu.async_copy(ad_hbm, ad_v, sem_st)
        pltpu.async_copy(m_hbm, m_v, sem_st)
        pltpu.async_copy(h_hbm.at[q], hq_v, sem_st)
        pltpu.async_copy(ep_hbm.at[pl.ds(base, CHK)], ev0, sem0)

        zeros = jnp.zeros((L,), jnp.float32)

        @pl.loop(0, NPAD // L, unroll=8)
        def _zd(i):
            den_v[0, pl.ds(i * L, L)] = zeros

        for r in range(DQ):
            @pl.loop(0, NPAD // L, unroll=8)
            def _zc(i, r=r):
                acc_v[r, pl.ds(i * L, L)] = zeros

        pltpu.make_async_copy(as_hbm, as_v, sem_st).wait()
        pltpu.make_async_copy(ad_hbm, ad_v, sem_st).wait()
        pltpu.make_async_copy(m_hbm, m_v, sem_st).wait()
        pltpu.make_async_copy(h_hbm.at[q], hq_v, sem_st).wait()

        m_vec = m_v[...]
        row_ids = [jnp.full((L,), r, jnp.int32) for r in range(DQ)]
        zero_ids = jnp.zeros((L,), jnp.int32)

        def _process(ev):
            @plsc.parallel_loop(0, CHK // L, unroll=4)
            def _vec(i):
                pk = ev[pl.ds(i * L, L)]
                s_idx = pk & 0x3FFF
                d_idx = lax.shift_right_logical(pk, 14)
                e = (plsc.load_gather(as_v, [s_idx])
                     + plsc.load_gather(ad_v, [d_idx]))
                w = jnp.exp(_leaky(e) - m_vec)

                @pl.when(q == 0)
                def _den():
                    plsc.addupdate_scatter(den_v, [zero_ids, d_idx], w)

                for r in range(DQ):
                    hv = plsc.load_gather(hq_v, [row_ids[r], s_idx])
                    plsc.addupdate_scatter(acc_v, [row_ids[r], d_idx], hv * w)

        @pl.loop(0, NCHKS // 2)
        def _chunk(cc):
            c0 = 2 * cc
            pltpu.make_async_copy(
                ep_hbm.at[pl.ds(base + c0 * CHK, CHK)], ev0, sem0).wait()
            pltpu.async_copy(
                ep_hbm.at[pl.ds(base + (c0 + 1) * CHK, CHK)], ev1, sem1)
            _process(ev0)
            pltpu.make_async_copy(
                ep_hbm.at[pl.ds(base + (c0 + 1) * CHK, CHK)], ev1, sem1).wait()

            @pl.when(cc != NCHKS // 2 - 1)
            def _prefetch():
                pltpu.async_copy(
                    ep_hbm.at[pl.ds(base + (c0 + 2) * CHK, CHK)], ev0, sem0)

            _process(ev1)

        pltpu.sync_copy(acc_v, out_hbm.at[wid])

        @pl.when(q == 0)
        def _pub_den():
            pltpu.sync_copy(den_v, den_hbm.at[gg])

    return edge_kernel(epk, a_s, a_d, m16, hT4)


def _tc1(xp, W1, a1s, a1d):
    def body(x_ref, w_ref, as_ref, ad_ref, h_ref, s_ref, d_ref, m_ref):
        h = jnp.dot(x_ref[...], w_ref[...], preferred_element_type=jnp.float32)
        hT = h.T
        h_ref[...] = hT
        a_s = jnp.dot(as_ref[...], hT, preferred_element_type=jnp.float32)
        a_d = jnp.dot(ad_ref[...], hT, preferred_element_type=jnp.float32)
        s_ref[...] = a_s
        d_ref[...] = a_d
        m_ref[...] = (jnp.max(a_s) + jnp.max(a_d)).reshape(1, 1)

    return pl.pallas_call(
        body,
        out_shape=(
            jax.ShapeDtypeStruct((D1, NPAD), jnp.float32),
            jax.ShapeDtypeStruct((1, NPAD), jnp.float32),
            jax.ShapeDtypeStruct((1, NPAD), jnp.float32),
            jax.ShapeDtypeStruct((1, 1), jnp.float32),
        ),
    )(xp, W1, a1s, a1d)


def _tc2(outp, denp, h1T, a_s1, a_d1, m1, b1, W2T, a2s, a2d):
    def body(o_ref, de_ref, h1_ref, s1_ref, d1_ref, m1_ref, b1_ref, w2_ref,
             as_ref, ad_ref, h2_ref, s_ref, d_ref, m_ref):
        w_self = jnp.exp(_leaky(s1_ref[...] + d1_ref[...]) - m1_ref[0, 0])
        op = o_ref[...].reshape(NC, NQ, NQ, DQ, NPAD)
        outT = op.sum(axis=(0, 1)).reshape(DP, NPAD)
        den = de_ref[...].sum(axis=0) + w_self + 1e-16
        numT = outT[:D1] + w_self * h1_ref[...]
        h1fT = jax.nn.relu(numT / den + b1_ref[...])
        h2T = jnp.dot(w2_ref[...], h1fT, preferred_element_type=jnp.float32)
        h2_ref[...] = jnp.concatenate(
            [h2T, jnp.zeros((DP - D2, NPAD), jnp.float32)], axis=0)
        a_s = jnp.dot(as_ref[...], h2T, preferred_element_type=jnp.float32)
        a_d = jnp.dot(ad_ref[...], h2T, preferred_element_type=jnp.float32)
        s_ref[...] = a_s
        d_ref[...] = a_d
        m_ref[...] = (jnp.max(a_s) + jnp.max(a_d)).reshape(1, 1)

    return pl.pallas_call(
        body,
        out_shape=(
            jax.ShapeDtypeStruct((DP, NPAD), jnp.float32),
            jax.ShapeDtypeStruct((1, NPAD), jnp.float32),
            jax.ShapeDtypeStruct((1, NPAD), jnp.float32),
            jax.ShapeDtypeStruct((1, 1), jnp.float32),
        ),
    )(outp, denp, h1T, a_s1, a_d1, m1, b1, W2T, a2s, a2d)


def _tc3(outp, denp, h2T, a_s2, a_d2, m2, b2, WfcT, bfc):
    def body(o_ref, de_ref, h2_ref, s2_ref, d2_ref, m2_ref, b2_ref, wf_ref,
             bf_ref, out_ref):
        w_self = jnp.exp(_leaky(s2_ref[...] + d2_ref[...]) - m2_ref[0, 0])
        op = o_ref[...].reshape(NC, NQ, NQ, DQ, NPAD)
        outT = op.sum(axis=(0, 1)).reshape(DP, NPAD)
        den = de_ref[...].sum(axis=0) + w_self + 1e-16
        numT = outT[:D2] + w_self * h2_ref[:D2, :]
        h2fT = jax.nn.relu(numT / den + b2_ref[...])
        out_ref[...] = jnp.dot(wf_ref[...], h2fT,
                               preferred_element_type=jnp.float32) + bf_ref[0, 0]

    return pl.pallas_call(
        body,
        out_shape=jax.ShapeDtypeStruct((1, NPAD), jnp.float32),
    )(outp, denp, h2T, a_s2, a_d2, m2, b2, WfcT, bfc)


def kernel(x, edge_index, W1, a1_src, a1_dst, b1, W2, a2_src, a2_dst, b2,
           Wfc, bfc):
    ei = edge_index.astype(jnp.int32)
    epk = (ei[1] << 14) | ei[0]
    xp = jnp.pad(x, ((0, NPAD - N), (0, 0)))

    h1T, a_s1, a_d1, m1 = _tc1(xp, W1, a1_src.reshape(1, D1),
                               a1_dst.reshape(1, D1))
    m1_16 = jnp.broadcast_to(m1.reshape(()), (L,))
    outp1, denp1 = _sc_edge(epk, a_s1.reshape(NPAD), a_d1.reshape(NPAD),
                            m1_16, h1T.reshape(NQ, DQ, NPAD))

    h2T, a_s2, a_d2, m2 = _tc2(outp1, denp1, h1T, a_s1, a_d1, m1,
                               b1.reshape(D1, 1), W2.T,
                               a2_src.reshape(1, D2), a2_dst.reshape(1, D2))
    m2_16 = jnp.broadcast_to(m2.reshape(()), (L,))
    outp2, denp2 = _sc_edge(epk, a_s2.reshape(NPAD), a_d2.reshape(NPAD),
                            m2_16, h2T.reshape(NQ, DQ, NPAD))

    out = _tc3(outp2, denp2, h2T, a_s2, a_d2, m2, b2.reshape(D2, 1),
               Wfc.T, bfc.reshape(1, 1))
    return out.reshape(NPAD, 1)[:N]
